# baseline (speedup 1.0000x reference)
import functools

import jax
import jax.numpy as jnp
from jax.experimental import pallas as pl
from jax.experimental.pallas import tpu as pltpu

T = 2048
H = 768
EH = 768
EI = 32
ES = 4
K = 16


def _router_kernel(x_ref, gwi_ref, gbi_ref, gws_ref, gbs_ref,
                   gi_ref, gs_ref, aux_ref):
    x = x_ref[...]
    li = jnp.dot(x, gwi_ref[...], preferred_element_type=jnp.float32)
    li = li + gbi_ref[...]
    p = jax.nn.softmax(li, axis=-1)

    iota = jax.lax.broadcasted_iota(jnp.int32, (T, EI), 1)
    rem = p
    sel = jnp.zeros((T, EI), dtype=jnp.bool_)
    for _ in range(K):
        m = jnp.max(rem, axis=-1, keepdims=True)
        ismax = rem == m
        first = jnp.min(jnp.where(ismax, iota, EI), axis=-1, keepdims=True)
        pick = iota == first
        sel = jnp.logical_or(sel, pick)
        rem = jnp.where(pick, -jnp.inf, rem)

    pv = jnp.where(sel, p, 0.0)
    gates_i = pv / jnp.sum(pv, axis=-1, keepdims=True)
    gi_ref[...] = gates_i

    density = jnp.mean(sel.astype(jnp.float32), axis=0)
    mean_prob = jnp.mean(p, axis=0)
    aux = jnp.float32(EI) * jnp.sum(density * mean_prob)
    aux_ref[...] = jnp.reshape(aux, (1, 1))

    ls = jnp.dot(x, gws_ref[...], preferred_element_type=jnp.float32)
    ls = ls + gbs_ref[...]
    gs_ref[...] = jax.nn.softmax(ls, axis=-1)


def _expert_kernel(num_e, x_ref, w1_ref, b1_ref, w2_ref, b2_ref, g_ref,
                   out_ref):
    e = pl.program_id(0)
    onehot = (jax.lax.broadcasted_iota(jnp.int32, (num_e, 1), 0) == e
              ).astype(jnp.float32)
    g = jnp.dot(g_ref[...], onehot,
                preferred_element_type=jnp.float32)
    xb = x_ref[...].astype(jnp.bfloat16)
    w1b = w1_ref[0].astype(jnp.bfloat16)
    h = jnp.dot(xb, w1b, preferred_element_type=jnp.float32)
    h = jax.nn.gelu(h + b1_ref[0])
    hb = (h * g).astype(jnp.bfloat16)
    w2b = w2_ref[0].astype(jnp.bfloat16)
    y = jnp.dot(hb, w2b, preferred_element_type=jnp.float32)
    contrib = y + g * b2_ref[0]

    @pl.when(e == 0)
    def _():
        out_ref[...] = contrib

    @pl.when(e > 0)
    def _():
        out_ref[...] += contrib


def _run_experts(x, w1, b1, w2, b2, gates, num_e):
    return pl.pallas_call(
        functools.partial(_expert_kernel, num_e),
        grid=(num_e,),
        in_specs=[
            pl.BlockSpec((T, H), lambda e: (0, 0)),
            pl.BlockSpec((1, H, EH), lambda e: (e, 0, 0)),
            pl.BlockSpec((1, 1, EH), lambda e: (e, 0, 0)),
            pl.BlockSpec((1, EH, H), lambda e: (e, 0, 0)),
            pl.BlockSpec((1, 1, H), lambda e: (e, 0, 0)),
            pl.BlockSpec((T, num_e), lambda e: (0, 0)),
        ],
        out_specs=pl.BlockSpec((T, H), lambda e: (0, 0)),
        out_shape=jax.ShapeDtypeStruct((T, H), jnp.float32),
    )(x, w1, b1, w2, b2, gates)


@jax.jit
def kernel(x, gw_i, gb_i, w1_i, b1_i, w2_i, b2_i,
           gw_s, gb_s, w1_s, b1_s, w2_s, b2_s):
    gates_i, gates_s, aux = pl.pallas_call(
        _router_kernel,
        out_shape=(
            jax.ShapeDtypeStruct((T, EI), jnp.float32),
            jax.ShapeDtypeStruct((T, ES), jnp.float32),
            jax.ShapeDtypeStruct((1, 1), jnp.float32),
        ),
    )(x, gw_i, gb_i.reshape(1, EI), gw_s, gb_s.reshape(1, ES))

    out_i = _run_experts(x, w1_i, b1_i.reshape(EI, 1, EH),
                         w2_i, b2_i.reshape(EI, 1, H), gates_i, EI)
    out_s = _run_experts(x, w1_s, b1_s.reshape(ES, 1, EH),
                         w2_s, b2_s.reshape(ES, 1, H), gates_s, ES)
    return out_i + out_s, aux[0, 0]


# speedup vs baseline: 1.0230x; 1.0230x over previous
import jax
import jax.numpy as jnp
from jax.experimental import pallas as pl
from jax.experimental.pallas import tpu as pltpu

T = 2048
H = 768
EH = 768
EI = 32
ES = 4
K = 16
E_ALL = EI + ES

_C1 = 0.7978845608028654
_C2 = _C1 * 0.044715


def _router_kernel(x_ref, gwi_ref, gbi_ref, gws_ref, gbs_ref,
                   gi_ref, gs_ref, aux_ref):
    x = x_ref[...]
    li = jnp.dot(x, gwi_ref[...], preferred_element_type=jnp.float32)
    li = li + gbi_ref[...]
    lt = li.T
    m0 = jnp.max(lt, axis=0, keepdims=True)
    ex = jnp.exp(lt - m0)
    p = ex / jnp.sum(ex, axis=0, keepdims=True)

    iota = jax.lax.broadcasted_iota(jnp.int32, (EI, T), 0)
    rem = p
    sel = jnp.zeros((EI, T), dtype=jnp.bool_)
    for _ in range(K):
        m = jnp.max(rem, axis=0, keepdims=True)
        ismax = rem == m
        first = jnp.min(jnp.where(ismax, iota, EI), axis=0, keepdims=True)
        pick = iota == first
        sel = jnp.logical_or(sel, pick)
        rem = jnp.where(pick, -jnp.inf, rem)

    pv = jnp.where(sel, p, 0.0)
    gates_i = pv / jnp.sum(pv, axis=0, keepdims=True)
    gi_ref[...] = gates_i.T

    density = jnp.mean(sel.astype(jnp.float32), axis=1)
    mean_prob = jnp.mean(p, axis=1)
    aux = jnp.float32(EI) * jnp.sum(density * mean_prob)
    aux_ref[...] = jnp.reshape(aux, (1, 1))

    ls = jnp.dot(x, gws_ref[...], preferred_element_type=jnp.float32)
    ls = (ls + gbs_ref[...]).T
    ms = jnp.max(ls, axis=0, keepdims=True)
    es_ = jnp.exp(ls - ms)
    gs_ref[...] = (es_ / jnp.sum(es_, axis=0, keepdims=True)).T


def _moe_kernel(xb_ref, w1i_ref, b1i_ref, w2i_ref, w1s_ref, b1s_ref,
                w2s_ref, b2c_ref, gc_ref, out_ref, h_ref):
    e = pl.program_id(0)

    @pl.when(e < E_ALL)
    def _phase_a():
        onehot = (jax.lax.broadcasted_iota(jnp.int32, (E_ALL, 1), 0) == e
                  ).astype(jnp.float32)
        g = jnp.dot(gc_ref[...], onehot,
                    preferred_element_type=jnp.float32)

        def mm1_gelu(w1_ref, b1_ref):
            z = jnp.dot(xb_ref[...], w1_ref[0].astype(jnp.bfloat16),
                        preferred_element_type=jnp.float32)
            z = z + b1_ref[0]
            z2 = z * z
            t = jnp.tanh((_C2 * z2 + _C1) * z)
            hw = 0.5 * (z * g)
            res = hw + hw * t
            h_ref[pl.ds(jax.lax.rem(e, 2), 1)] = (
                res.astype(jnp.bfloat16).reshape(1, T, EH))

        @pl.when(e < EI)
        def _():
            mm1_gelu(w1i_ref, b1i_ref)

        @pl.when(e >= EI)
        def _():
            mm1_gelu(w1s_ref, b1s_ref)

    @pl.when(e >= 1)
    def _phase_b():
        hp = h_ref[pl.ds(jax.lax.rem(e - 1, 2), 1)][0]

        @pl.when(e - 1 < EI)
        def _():
            y = jnp.dot(hp, w2i_ref[0].astype(jnp.bfloat16),
                        preferred_element_type=jnp.float32)

            @pl.when(e == 1)
            def _(y=y):
                out_ref[...] = y

            @pl.when(e > 1)
            def _(y=y):
                out_ref[...] += y

        @pl.when(e - 1 >= EI)
        def _():
            y = jnp.dot(hp, w2s_ref[0].astype(jnp.bfloat16),
                        preferred_element_type=jnp.float32)
            out_ref[...] += y

    @pl.when(e == E_ALL)
    def _tail():
        out_ref[...] += jnp.dot(gc_ref[...], b2c_ref[...],
                                preferred_element_type=jnp.float32)


@jax.jit
def kernel(x, gw_i, gb_i, w1_i, b1_i, w2_i, b2_i,
           gw_s, gb_s, w1_s, b1_s, w2_s, b2_s):
    gates_i, gates_s, aux = pl.pallas_call(
        _router_kernel,
        out_shape=(
            jax.ShapeDtypeStruct((T, EI), jnp.float32),
            jax.ShapeDtypeStruct((T, ES), jnp.float32),
            jax.ShapeDtypeStruct((1, 1), jnp.float32),
        ),
    )(x, gw_i, gb_i.reshape(1, EI), gw_s, gb_s.reshape(1, ES))

    xb = x.astype(jnp.bfloat16)
    gc = jnp.concatenate([gates_i, gates_s], axis=1)
    b2c = jnp.concatenate([b2_i, b2_s], axis=0)

    out = pl.pallas_call(
        _moe_kernel,
        grid=(E_ALL + 1,),
        in_specs=[
            pl.BlockSpec((T, H), lambda e: (0, 0)),
            pl.BlockSpec((1, H, EH),
                         lambda e: (jnp.minimum(e, EI - 1), 0, 0)),
            pl.BlockSpec((1, 1, EH),
                         lambda e: (jnp.minimum(e, EI - 1), 0, 0)),
            pl.BlockSpec((1, EH, H),
                         lambda e: (jnp.clip(e - 1, 0, EI - 1), 0, 0)),
            pl.BlockSpec((1, H, EH),
                         lambda e: (jnp.clip(e - EI, 0, ES - 1), 0, 0)),
            pl.BlockSpec((1, 1, EH),
                         lambda e: (jnp.clip(e - EI, 0, ES - 1), 0, 0)),
            pl.BlockSpec((1, EH, H),
                         lambda e: (jnp.clip(e - EI - 1, 0, ES - 1), 0, 0)),
            pl.BlockSpec((E_ALL, H), lambda e: (0, 0)),
            pl.BlockSpec((T, E_ALL), lambda e: (0, 0)),
        ],
        out_specs=pl.BlockSpec((T, H), lambda e: (0, 0)),
        out_shape=jax.ShapeDtypeStruct((T, H), jnp.float32),
        scratch_shapes=[pltpu.VMEM((2, T, EH), jnp.bfloat16)],
    )(xb, w1_i, b1_i.reshape(EI, 1, EH), w2_i,
      w1_s, b1_s.reshape(ES, 1, EH), w2_s, b2c, gc)

    return out, aux[0, 0]


# speedup vs baseline: 1.0547x; 1.0310x over previous
import functools

import jax
import jax.numpy as jnp
from jax.experimental import pallas as pl
from jax.experimental.pallas import tpu as pltpu

T = 2048
H = 768
EH = 768
EI = 32
ES = 4
K = 16
E_ALL = EI + ES

_C1 = 0.7978845608028654
_C2 = _C1 * 0.044715


def _router_kernel(x_ref, gwi_ref, gbi_ref, gws_ref, gbs_ref,
                   gi_ref, gs_ref, aux_ref):
    x = x_ref[...]
    li = jnp.dot(x, gwi_ref[...], preferred_element_type=jnp.float32)
    li = li + gbi_ref[...]
    lt = li.T
    m0 = jnp.max(lt, axis=0, keepdims=True)
    ex = jnp.exp(lt - m0)
    p = ex / jnp.sum(ex, axis=0, keepdims=True)

    iota = jax.lax.broadcasted_iota(jnp.int32, (EI, T), 0)
    rem = p
    sel = jnp.zeros((EI, T), dtype=jnp.bool_)
    for _ in range(K):
        m = jnp.max(rem, axis=0, keepdims=True)
        ismax = rem == m
        first = jnp.min(jnp.where(ismax, iota, EI), axis=0, keepdims=True)
        pick = iota == first
        sel = jnp.logical_or(sel, pick)
        rem = jnp.where(pick, -jnp.inf, rem)

    pv = jnp.where(sel, p, 0.0)
    gates_i = pv / jnp.sum(pv, axis=0, keepdims=True)
    gi_ref[...] = gates_i.T

    density = jnp.mean(sel.astype(jnp.float32), axis=1)
    mean_prob = jnp.mean(p, axis=1)
    aux = jnp.float32(EI) * jnp.sum(density * mean_prob)
    aux_ref[...] = jnp.reshape(aux, (1, 1))

    ls = jnp.dot(x, gws_ref[...], preferred_element_type=jnp.float32)
    ls = (ls + gbs_ref[...]).T
    ms = jnp.max(ls, axis=0, keepdims=True)
    es_ = jnp.exp(ls - ms)
    gs_ref[...] = (es_ / jnp.sum(es_, axis=0, keepdims=True)).T


def _pipe_body(num_e, e, x_ref, w1_ref, b1_ref, w2_ref, g_ref, out_ref,
               h_ref):
    wp = jax.lax.rem(e, 2)
    rp = jax.lax.rem(e + 1, 2)

    hp = h_ref[pl.ds(rp, 1)][0]
    y = jnp.dot(hp, w2_ref[0], preferred_element_type=jnp.float32)
    out_ref[...] += y

    onehot = (jax.lax.broadcasted_iota(jnp.int32, (num_e, 1), 0) == e
              ).astype(jnp.float32)
    g = jnp.dot(g_ref[...], onehot,
                preferred_element_type=jnp.float32)
    z = jnp.dot(x_ref[...], w1_ref[0], preferred_element_type=jnp.float32)
    z = z + b1_ref[0]
    t = jnp.tanh((_C2 * (z * z) + _C1) * z)
    hw = 0.5 * (z * g)
    res = hw + hw * t
    h_ref[pl.ds(wp, 1)] = res.reshape(1, T, EH)


def _expert_i_kernel(x_ref, w1_ref, b1_ref, w2_ref, g_ref, out_ref, h_ref):
    e = pl.program_id(0)

    @pl.when(e == 0)
    def _():
        h_ref[...] = jnp.zeros((2, T, EH), jnp.float32)
        out_ref[...] = jnp.zeros((T, H), jnp.float32)

    _pipe_body(EI, e, x_ref, w1_ref, b1_ref, w2_ref, g_ref, out_ref, h_ref)


def _expert_s_kernel(prev_ref, x_ref, w1_ref, b1_ref, w2_ref, g_ref,
                     gc_ref, b2c_ref, out_ref, h_ref):
    e = pl.program_id(0)

    @pl.when(e == 0)
    def _():
        h_ref[...] = jnp.zeros((2, T, EH), jnp.float32)
        out_ref[...] = prev_ref[...]

    _pipe_body(ES, e, x_ref, w1_ref, b1_ref, w2_ref, g_ref, out_ref, h_ref)

    @pl.when(e == ES)
    def _():
        out_ref[...] += jnp.dot(gc_ref[...], b2c_ref[...],
                                preferred_element_type=jnp.float32)


@jax.jit
def kernel(x, gw_i, gb_i, w1_i, b1_i, w2_i, b2_i,
           gw_s, gb_s, w1_s, b1_s, w2_s, b2_s):
    gates_i, gates_s, aux = pl.pallas_call(
        _router_kernel,
        out_shape=(
            jax.ShapeDtypeStruct((T, EI), jnp.float32),
            jax.ShapeDtypeStruct((T, ES), jnp.float32),
            jax.ShapeDtypeStruct((1, 1), jnp.float32),
        ),
    )(x, gw_i, gb_i.reshape(1, EI), gw_s, gb_s.reshape(1, ES))

    out_i = pl.pallas_call(
        _expert_i_kernel,
        grid=(EI + 1,),
        in_specs=[
            pl.BlockSpec((T, H), lambda e: (0, 0)),
            pl.BlockSpec((1, H, EH),
                         lambda e: (jnp.minimum(e, EI - 1), 0, 0)),
            pl.BlockSpec((1, 1, EH),
                         lambda e: (jnp.minimum(e, EI - 1), 0, 0)),
            pl.BlockSpec((1, EH, H),
                         lambda e: (jnp.clip(e - 1, 0, EI - 1), 0, 0)),
            pl.BlockSpec((T, EI), lambda e: (0, 0)),
        ],
        out_specs=pl.BlockSpec((T, H), lambda e: (0, 0)),
        out_shape=jax.ShapeDtypeStruct((T, H), jnp.float32),
        scratch_shapes=[pltpu.VMEM((2, T, EH), jnp.float32)],
    )(x, w1_i, b1_i.reshape(EI, 1, EH), w2_i, gates_i)

    gc = jnp.concatenate([gates_i, gates_s], axis=1)
    b2c = jnp.concatenate([b2_i, b2_s], axis=0)

    out = pl.pallas_call(
        _expert_s_kernel,
        grid=(ES + 1,),
        in_specs=[
            pl.BlockSpec((T, H), lambda e: (0, 0)),
            pl.BlockSpec((T, H), lambda e: (0, 0)),
            pl.BlockSpec((1, H, EH),
                         lambda e: (jnp.minimum(e, ES - 1), 0, 0)),
            pl.BlockSpec((1, 1, EH),
                         lambda e: (jnp.minimum(e, ES - 1), 0, 0)),
            pl.BlockSpec((1, EH, H),
                         lambda e: (jnp.clip(e - 1, 0, ES - 1), 0, 0)),
            pl.BlockSpec((T, ES), lambda e: (0, 0)),
            pl.BlockSpec((T, E_ALL), lambda e: (0, 0)),
            pl.BlockSpec((E_ALL, H), lambda e: (0, 0)),
        ],
        out_specs=pl.BlockSpec((T, H), lambda e: (0, 0)),
        out_shape=jax.ShapeDtypeStruct((T, H), jnp.float32),
        scratch_shapes=[pltpu.VMEM((2, T, EH), jnp.float32)],
        input_output_aliases={0: 0},
    )(out_i, x, w1_s, b1_s.reshape(ES, 1, EH), w2_s, gates_s, gc, b2c)

    return out, aux[0, 0]


# speedup vs baseline: 1.1407x; 1.0816x over previous
import functools

import jax
import jax.numpy as jnp
from jax.experimental import pallas as pl
from jax.experimental.pallas import tpu as pltpu

T = 2048
H = 768
EH = 768
EI = 32
ES = 4
K = 16
E_ALL = EI + ES

_C1 = 0.7978845608028654
_C2 = _C1 * 0.044715


def _router_kernel(x_ref, gwi_ref, gbi_ref, gws_ref, gbs_ref,
                   gi_ref, gs_ref, aux_ref):
    x = x_ref[...]
    li = jnp.dot(x, gwi_ref[...], preferred_element_type=jnp.float32)
    li = li + gbi_ref[...]
    lt = li.T
    m0 = jnp.max(lt, axis=0, keepdims=True)
    ex = jnp.exp(lt - m0)
    p = ex / jnp.sum(ex, axis=0, keepdims=True)

    iota = jax.lax.broadcasted_iota(jnp.int32, (EI, T), 0)
    rem = p
    sel = jnp.zeros((EI, T), dtype=jnp.bool_)
    for _ in range(K):
        m = jnp.max(rem, axis=0, keepdims=True)
        ismax = rem == m
        first = jnp.min(jnp.where(ismax, iota, EI), axis=0, keepdims=True)
        pick = iota == first
        sel = jnp.logical_or(sel, pick)
        rem = jnp.where(pick, -jnp.inf, rem)

    pv = jnp.where(sel, p, 0.0)
    gates_i = pv / jnp.sum(pv, axis=0, keepdims=True)
    gi_ref[...] = gates_i.T

    density = jnp.mean(sel.astype(jnp.float32), axis=1)
    mean_prob = jnp.mean(p, axis=1)
    aux = jnp.float32(EI) * jnp.sum(density * mean_prob)
    aux_ref[...] = jnp.reshape(aux, (1, 1))

    ls = jnp.dot(x, gws_ref[...], preferred_element_type=jnp.float32)
    ls = (ls + gbs_ref[...]).T
    ms = jnp.max(ls, axis=0, keepdims=True)
    es_ = jnp.exp(ls - ms)
    gs_ref[...] = (es_ / jnp.sum(es_, axis=0, keepdims=True)).T


def _mlp_contrib(num_e, e, x_ref, w1_ref, b1_ref, w2_ref, g_ref):
    onehot = (jax.lax.broadcasted_iota(jnp.int32, (num_e, 1), 0) == e
              ).astype(jnp.float32)
    g = jnp.dot(g_ref[...], onehot,
                preferred_element_type=jnp.float32)
    z = jnp.dot(x_ref[...], w1_ref[0], preferred_element_type=jnp.float32)
    z = z + b1_ref[0]
    t = jnp.tanh((_C2 * (z * z) + _C1) * z)
    hw = 0.5 * (z * g)
    h = hw + hw * t
    return jnp.dot(h, w2_ref[0], preferred_element_type=jnp.float32)


def _expert_i_kernel(x_ref, w1_ref, b1_ref, w2_ref, g_ref, out_ref):
    e = pl.program_id(0)
    y = _mlp_contrib(EI, e, x_ref, w1_ref, b1_ref, w2_ref, g_ref)

    @pl.when(e == 0)
    def _():
        out_ref[...] = y

    @pl.when(e > 0)
    def _():
        out_ref[...] += y


def _expert_s_kernel(prev_ref, x_ref, w1_ref, b1_ref, w2_ref, g_ref,
                     gc_ref, b2c_ref, out_ref):
    e = pl.program_id(0)
    y = _mlp_contrib(ES, e, x_ref, w1_ref, b1_ref, w2_ref, g_ref)

    @pl.when(e == 0)
    def _():
        out_ref[...] = prev_ref[...] + y

    @pl.when(e > 0)
    def _():
        out_ref[...] += y

    @pl.when(e == ES - 1)
    def _():
        out_ref[...] += jnp.dot(gc_ref[...], b2c_ref[...],
                                preferred_element_type=jnp.float32)


@jax.jit
def kernel(x, gw_i, gb_i, w1_i, b1_i, w2_i, b2_i,
           gw_s, gb_s, w1_s, b1_s, w2_s, b2_s):
    gates_i, gates_s, aux = pl.pallas_call(
        _router_kernel,
        out_shape=(
            jax.ShapeDtypeStruct((T, EI), jnp.float32),
            jax.ShapeDtypeStruct((T, ES), jnp.float32),
            jax.ShapeDtypeStruct((1, 1), jnp.float32),
        ),
    )(x, gw_i, gb_i.reshape(1, EI), gw_s, gb_s.reshape(1, ES))

    out_i = pl.pallas_call(
        _expert_i_kernel,
        grid=(EI,),
        in_specs=[
            pl.BlockSpec((T, H), lambda e: (0, 0)),
            pl.BlockSpec((1, H, EH), lambda e: (e, 0, 0)),
            pl.BlockSpec((1, 1, EH), lambda e: (e, 0, 0)),
            pl.BlockSpec((1, EH, H), lambda e: (e, 0, 0)),
            pl.BlockSpec((T, EI), lambda e: (0, 0)),
        ],
        out_specs=pl.BlockSpec((T, H), lambda e: (0, 0)),
        out_shape=jax.ShapeDtypeStruct((T, H), jnp.float32),
    )(x, w1_i, b1_i.reshape(EI, 1, EH), w2_i, gates_i)

    gc = jnp.concatenate([gates_i, gates_s], axis=1)
    b2c = jnp.concatenate([b2_i, b2_s], axis=0)

    out = pl.pallas_call(
        _expert_s_kernel,
        grid=(ES,),
        in_specs=[
            pl.BlockSpec((T, H), lambda e: (0, 0)),
            pl.BlockSpec((T, H), lambda e: (0, 0)),
            pl.BlockSpec((1, H, EH), lambda e: (e, 0, 0)),
            pl.BlockSpec((1, 1, EH), lambda e: (e, 0, 0)),
            pl.BlockSpec((1, EH, H), lambda e: (e, 0, 0)),
            pl.BlockSpec((T, ES), lambda e: (0, 0)),
            pl.BlockSpec((T, E_ALL), lambda e: (0, 0)),
            pl.BlockSpec((E_ALL, H), lambda e: (0, 0)),
        ],
        out_specs=pl.BlockSpec((T, H), lambda e: (0, 0)),
        out_shape=jax.ShapeDtypeStruct((T, H), jnp.float32),
        input_output_aliases={0: 0},
    )(out_i, x, w1_s, b1_s.reshape(ES, 1, EH), w2_s, gates_s, gc, b2c)

    return out, aux[0, 0]
